# Initial kernel scaffold; baseline (speedup 1.0000x reference)
import jax
import jax.numpy as jnp
from jax.experimental import pallas as pl


def kernel(x, edge_index, edge_attr, batch, Wx, bx, We, be, W1, b1, W2, b2, bn_g, bn_b, Wl1, bl1, bnf_g, bnf_b, Wl2, bl2):
    raise NotImplementedError("write your pallas kernel here")


# speedup vs baseline: 1.6252x; 1.6252x over previous
import dataclasses

import jax
import jax.numpy as jnp
from jax import lax
from jax.experimental import pallas as pl
from jax.experimental.pallas import tpu as pltpu
from jax.experimental.pallas import tpu_sc as plsc

_N = 10000
_E = 320000
_H = 128
_G = 64

_NT = 32
_RB = 312
_TRASH = 332
_ACC_R = 336
_CAP = 12000
_C = 80
_DCH = 2560
_MAGIC, _SHIFT = 6722, 21


def _sc_params():
    cp = pltpu.CompilerParams()
    if "needs_layout_passes" in pltpu.CompilerParams.__dataclass_fields__:
        cp = dataclasses.replace(cp, needs_layout_passes=False)
    return cp


def _sc_partition(src, dst):
    mesh = plsc.VectorSubcoreMesh(core_axis_name="c", subcore_axis_name="s")

    def body(src_hbm, dst_hbm, lists_hbm, cnt_hbm,
             sstage, dstage, elist, slist, dlist, cnt_v, pos_ref):
        w = lax.axis_index("c") * 16 + lax.axis_index("s")
        pos_ref[0] = 0
        lane = lax.iota(jnp.int32, 16)

        @pl.loop(0, _E // _DCH)
        def _blk(b):
            pltpu.sync_copy(src_hbm.at[pl.ds(b * _DCH, _DCH)], sstage)
            pltpu.sync_copy(dst_hbm.at[pl.ds(b * _DCH, _DCH)], dstage)

            @pl.loop(0, _DCH // 16)
            def _grp(g):
                dv = dstage[pl.ds(g * 16, 16)]
                sv = sstage[pl.ds(g * 16, 16)]
                eidv = lane + (b * _DCH + g * 16)
                ow = jnp.minimum((dv * _MAGIC) >> _SHIFT, 31)
                m = ow == w
                pos = pos_ref[0]
                plsc.store_compressed(elist.at[pl.ds(pos, 16)], eidv, mask=m)
                plsc.store_compressed(slist.at[pl.ds(pos, 16)], sv, mask=m)
                plsc.store_compressed(dlist.at[pl.ds(pos, 16)],
                                      dv - w * _RB, mask=m)
                pos_ref[0] = pos + jnp.sum(m.astype(jnp.int32))

        cnt = pos_ref[0]
        padded = ((cnt + (_C - 1)) // _C) * _C
        zero16 = jnp.zeros((16,), jnp.int32)
        trash16 = jnp.full((16,), _TRASH, jnp.int32)
        for k in range(_C // 16):
            elist[pl.ds(cnt + k * 16, 16)] = zero16
            slist[pl.ds(cnt + k * 16, 16)] = zero16
            dlist[pl.ds(cnt + k * 16, 16)] = trash16

        pltpu.sync_copy(elist, lists_hbm.at[0, w])
        pltpu.sync_copy(slist, lists_hbm.at[1, w])
        pltpu.sync_copy(dlist, lists_hbm.at[2, w])
        cnt_v[...] = jnp.broadcast_to((padded // _C)[None], (16,))
        pltpu.sync_copy(cnt_v, cnt_hbm.at[w])

    return pl.kernel(
        body,
        out_type=[jax.ShapeDtypeStruct((3, _NT, _CAP), jnp.int32),
                  jax.ShapeDtypeStruct((_NT, 16), jnp.int32)],
        mesh=mesh,
        compiler_params=_sc_params(),
        scratch_types=[
            pltpu.VMEM((_DCH,), jnp.int32),
            pltpu.VMEM((_DCH,), jnp.int32),
            pltpu.VMEM((_CAP,), jnp.int32),
            pltpu.VMEM((_CAP,), jnp.int32),
            pltpu.VMEM((_CAP,), jnp.int32),
            pltpu.VMEM((16,), jnp.int32),
            pltpu.SMEM((1,), jnp.int32),
        ],
    )(src, dst)


def _sc_layer(h, ea, lists, cnts, zeros):
    mesh = plsc.VectorSubcoreMesh(core_axis_name="c", subcore_axis_name="s")

    def body(h_hbm, ea_hbm, lists_hbm, cnt_hbm, z_hbm, out_hbm,
             elist, slist, dlist, cnt_v, hbuf, ebuf, acc):
        w = lax.axis_index("c") * 16 + lax.axis_index("s")
        pltpu.sync_copy(lists_hbm.at[0, w], elist)
        pltpu.sync_copy(lists_hbm.at[1, w], slist)
        pltpu.sync_copy(lists_hbm.at[2, w], dlist)
        pltpu.sync_copy(cnt_hbm.at[w], cnt_v)
        pltpu.sync_copy(z_hbm.at[pl.ds(0, _ACC_R)], acc)
        nch = cnt_v[pl.ds(0, 16)][0]

        @pl.loop(0, nch)
        def _chunk(i):
            eidx = elist.at[pl.ds(i * _C, _C)]
            sidx = slist.at[pl.ds(i * _C, _C)]
            pltpu.sync_copy(ea_hbm.at[eidx], ebuf)
            pltpu.sync_copy(h_hbm.at[sidx], hbuf)

            @pl.loop(0, _C // 16)
            def _grp(g):
                dlv = dlist[pl.ds(i * _C + g * 16, 16)]
                for r16 in range(16):
                    r = g * 16 + r16
                    dl = dlv[r16]
                    for j in range(_H // 16):
                        sl = pl.ds(j * 16, 16)
                        v = jnp.maximum(hbuf[r, sl] + ebuf[r, sl], 0.0)
                        acc[dl, sl] = acc[dl, sl] + v

        pltpu.sync_copy(acc.at[pl.ds(0, _RB)], out_hbm.at[pl.ds(w * _RB, _RB)])

        @pl.when(w == 31)
        def _():
            pltpu.sync_copy(acc.at[pl.ds(_RB, 16)],
                            out_hbm.at[pl.ds(_NT * _RB, 16)])

    return pl.kernel(
        body,
        out_type=jax.ShapeDtypeStruct((_N, _H), jnp.float32),
        mesh=mesh,
        compiler_params=_sc_params(),
        scratch_types=[
            pltpu.VMEM((_CAP,), jnp.int32),
            pltpu.VMEM((_CAP,), jnp.int32),
            pltpu.VMEM((_CAP,), jnp.int32),
            pltpu.VMEM((16,), jnp.int32),
            pltpu.VMEM((_C, _H), jnp.float32),
            pltpu.VMEM((_C, _H), jnp.float32),
            pltpu.VMEM((_ACC_R, _H), jnp.float32),
        ],
    )(h, ea, lists, cnts, zeros)


def _matmul_bias_body(a_ref, w_ref, b_ref, o_ref):
    o_ref[...] = (
        jnp.dot(a_ref[...], w_ref[...], preferred_element_type=jnp.float32)
        + b_ref[...]
    )


def _tc_h0(x, Wx, bx2):
    return pl.pallas_call(
        _matmul_bias_body,
        out_shape=jax.ShapeDtypeStruct((_N, _H), jnp.float32),
    )(x, Wx, bx2)


def _tc_ea(eattr, We, be2):
    be_rows = 2000
    ed = eattr.shape[1]
    return pl.pallas_call(
        _matmul_bias_body,
        grid=(_E // be_rows,),
        in_specs=[
            pl.BlockSpec((be_rows, ed), lambda i: (i, 0)),
            pl.BlockSpec((ed, _H), lambda i: (0, 0)),
            pl.BlockSpec((1, _H), lambda i: (0, 0)),
        ],
        out_specs=pl.BlockSpec((be_rows, _H), lambda i: (i, 0)),
        out_shape=jax.ShapeDtypeStruct((_E, _H), jnp.float32),
    )(eattr, We, be2)


def _colsum_2half(u_ref, scratch):
    nt = _N // 8

    def half(lo):
        def it(t, a):
            return a + u_ref[pl.ds((lo + t) * 8, 8), :]
        return lax.fori_loop(0, nt // 2, it,
                             jnp.zeros((8, _H), jnp.float32))

    acc = half(0) + half(nt // 2)
    s4 = acc[:4] + acc[4:]
    s2 = s4[:2] + s4[2:]
    return s2[:1] + s2[1:]


def _tc_layer(h, aggr, W1i, b1i, W2i, b2i, g_i, bb_i):
    def body(h_ref, a_ref, w1_ref, b1_ref, w2_ref, b2_ref, g_ref, bb_ref,
             o_ref, u_ref, d_ref):
        z = h_ref[...] + a_ref[...]
        t = jnp.maximum(
            jnp.dot(z, w1_ref[...], preferred_element_type=jnp.float32)
            + b1_ref[...], 0.0)
        u_ref[...] = (
            jnp.dot(t, w2_ref[...], preferred_element_type=jnp.float32)
            + b2_ref[...])
        mu = _colsum_2half(u_ref, None) / jnp.float32(_N)
        d_ref[...] = (u_ref[...] - mu) ** 2
        var = _colsum_2half(d_ref, None) / jnp.float32(_N)
        zn = (u_ref[...] - mu) / jnp.sqrt(var + 1e-5) * g_ref[...] + bb_ref[...]
        o_ref[...] = h_ref[...] + jnp.maximum(zn, 0.0)

    return pl.pallas_call(
        body,
        out_shape=jax.ShapeDtypeStruct((_N, _H), jnp.float32),
        scratch_shapes=[pltpu.VMEM((_N, _H), jnp.float32),
                        pltpu.VMEM((_N, _H), jnp.float32)],
    )(h, aggr, W1i, b1i, W2i, b2i, g_i, bb_i)


def _tc_head(h, batch2, Wl1, bl1_2, g2, b2_, Wl2p, bl2p):
    def body(h_ref, b_ref, w1_ref, b1_ref, g_ref, bb_ref, w2_ref, b2_ref,
             o_ref):
        seg = b_ref[...]
        ids = lax.broadcasted_iota(jnp.int32, (_G, _N), 0)
        mask = (ids == seg).astype(jnp.float32)
        psum = jnp.dot(mask, h_ref[...], preferred_element_type=jnp.float32,
                       precision=lax.Precision.HIGHEST)
        cnt = jnp.sum(mask, axis=1, keepdims=True)
        pooled = psum / jnp.maximum(cnt, 1.0)
        o = (jnp.dot(pooled, w1_ref[...], preferred_element_type=jnp.float32,
                     precision=lax.Precision.HIGHEST)
             + b1_ref[...])
        mu = jnp.mean(o, axis=0, keepdims=True)
        var = jnp.mean((o - mu) ** 2, axis=0, keepdims=True)
        o = (o - mu) / jnp.sqrt(var + 1e-5) * g_ref[...] + bb_ref[...]
        o = jnp.maximum(o, 0.0)
        o_ref[...] = (
            jnp.dot(o, w2_ref[...], preferred_element_type=jnp.float32,
                    precision=lax.Precision.HIGHEST)
            + b2_ref[...])

    return pl.pallas_call(
        body,
        out_shape=jax.ShapeDtypeStruct((_G, _H), jnp.float32),
    )(h, batch2, Wl1, bl1_2, g2, b2_, Wl2p, bl2p)


def kernel(x, edge_index, edge_attr, batch, Wx, bx, We, be, W1, b1, W2, b2,
           bn_g, bn_b, Wl1, bl1, bnf_g, bnf_b, Wl2, bl2):
    src = edge_index[0].astype(jnp.int32)
    dst = edge_index[1].astype(jnp.int32)
    out_dim = Wl2.shape[1]

    h = _tc_h0(x, Wx, bx.reshape(1, -1))
    ea = _tc_ea(edge_attr, We, be.reshape(1, -1))
    lists, cnts = _sc_partition(src, dst)
    zeros = jnp.zeros((_N, _H), jnp.float32)

    for i in range(W1.shape[0]):
        aggr = _sc_layer(h, ea, lists, cnts, zeros)
        h = _tc_layer(h, aggr, W1[i], b1[i].reshape(1, -1), W2[i],
                      b2[i].reshape(1, -1), bn_g[i].reshape(1, -1),
                      bn_b[i].reshape(1, -1))

    Wl2p = jnp.zeros((_H, _H), jnp.float32).at[:, :out_dim].set(Wl2)
    bl2p = jnp.zeros((1, _H), jnp.float32).at[0, :out_dim].set(bl2)
    o = _tc_head(h, batch.reshape(1, -1).astype(jnp.int32), Wl1,
                 bl1.reshape(1, -1), bnf_g.reshape(1, -1),
                 bnf_b.reshape(1, -1), Wl2p, bl2p)
    return o[:, :out_dim]
